# Initial kernel scaffold; baseline (speedup 1.0000x reference)
import functools

import jax
import jax.numpy as jnp
from jax import lax
from jax.experimental import pallas as pl
from jax.experimental.pallas import tpu as pltpu
from jax.experimental.pallas import tpu_sc as plsc

NN = 10000
DD = 128
EPS = 1e-5

NC = 2
NS = 16
BLK = 128
NBLK = 80
EPT = NBLK * BLK
CAP = NC * NS * EPT
NPAD = 10240
ROWS_PER_TILE = NN // NS
ZROWS = 32
DEGW = 16

_mesh = plsc.VectorSubcoreMesh(
    core_axis_name="c", subcore_axis_name="s", num_cores=NC, num_subcores=NS)


@functools.partial(
    pl.kernel,
    out_type=jax.ShapeDtypeStruct((NC, NN, DEGW), jnp.float32),
    mesh=_mesh,
    scratch_types=[
        pltpu.VMEM_SHARED((NPAD, DEGW), jnp.float32),
        pltpu.VMEM((NBLK, BLK), jnp.int32),
        pltpu.VMEM((BLK, DEGW), jnp.float32),
        pltpu.VMEM((BLK, DEGW), jnp.float32),
    ],
)
def _deg_kernel(dst_hbm, out_hbm, deg_s, dst_v, obuf, zbuf):
    c = lax.axis_index("c")
    s = lax.axis_index("s")
    ones16 = jnp.full((16,), 1.0, jnp.float32)
    zeros16 = jnp.zeros((16,), jnp.float32)

    def fill(i, _):
        obuf[i, :] = ones16
        zbuf[i, :] = zeros16
        return 0
    lax.fori_loop(0, BLK, fill, 0)

    rows_pt = NPAD // NS
    zbase = s * rows_pt

    def zloop(i, _):
        pltpu.sync_copy(zbuf, deg_s.at[pl.ds(zbase + i * BLK, BLK)])
        return 0
    lax.fori_loop(0, rows_pt // BLK, zloop, 0)
    plsc.subcore_barrier()

    pltpu.sync_copy(dst_hbm.at[c, s], dst_v)

    def sloop(j, _):
        pltpu.sync_copy(obuf, deg_s.at[dst_v.at[j]], add=True)
        return 0
    lax.fori_loop(0, NBLK, sloop, 0)
    plsc.subcore_barrier()

    wbase = s * ROWS_PER_TILE
    pltpu.sync_copy(deg_s.at[pl.ds(wbase, ROWS_PER_TILE)],
                    out_hbm.at[c, pl.ds(wbase, ROWS_PER_TILE)])


@functools.partial(
    pl.kernel,
    out_type=jax.ShapeDtypeStruct((NC, NN, DD), jnp.float32),
    mesh=_mesh,
    scratch_types=[
        pltpu.VMEM_SHARED((NPAD, DD), jnp.float32),
        pltpu.VMEM((NBLK, BLK), jnp.int32),
        pltpu.VMEM((NBLK, BLK), jnp.int32),
        pltpu.VMEM((BLK, DD), jnp.float32),
        pltpu.VMEM((BLK, DD), jnp.float32),
        pltpu.VMEM((ZROWS, DD), jnp.float32),
        pltpu.SemaphoreType.DMA,
        pltpu.SemaphoreType.DMA,
    ],
)
def _agg_kernel(y_hbm, src_hbm, dst_hbm, out_hbm,
                agg_s, src_v, dst_v, gb0, gb1, zbuf, sem0, sem1):
    c = lax.axis_index("c")
    s = lax.axis_index("s")
    zeros16 = jnp.zeros((16,), jnp.float32)
    lanes_per_row = DD // 16

    def zfill(i, _):
        r = i // lanes_per_row
        k = (i % lanes_per_row) * 16
        zbuf[r, pl.ds(k, 16)] = zeros16
        return 0
    lax.fori_loop(0, ZROWS * lanes_per_row, zfill, 0)

    rows_pt = NPAD // NS
    zbase = s * rows_pt

    def zloop(i, _):
        pltpu.sync_copy(zbuf, agg_s.at[pl.ds(zbase + i * ZROWS, ZROWS)])
        return 0
    lax.fori_loop(0, rows_pt // ZROWS, zloop, 0)
    plsc.subcore_barrier()

    pltpu.sync_copy(src_hbm.at[c, s], src_v)
    pltpu.sync_copy(dst_hbm.at[c, s], dst_v)

    pltpu.async_copy(y_hbm.at[src_v.at[0]], gb0, sem0)
    pltpu.async_copy(y_hbm.at[src_v.at[1]], gb1, sem1)

    def eloop(i, _):
        j0 = 2 * i
        pltpu.make_async_copy(y_hbm.at[src_v.at[j0]], gb0, sem0).wait()
        pltpu.sync_copy(gb0, agg_s.at[dst_v.at[j0]], add=True)
        pltpu.async_copy(y_hbm.at[src_v.at[j0 + 2]], gb0, sem0)
        pltpu.make_async_copy(y_hbm.at[src_v.at[j0 + 1]], gb1, sem1).wait()
        pltpu.sync_copy(gb1, agg_s.at[dst_v.at[j0 + 1]], add=True)
        pltpu.async_copy(y_hbm.at[src_v.at[j0 + 3]], gb1, sem1)
        return 0
    lax.fori_loop(0, NBLK // 2 - 1, eloop, 0)

    pltpu.make_async_copy(y_hbm.at[src_v.at[NBLK - 2]], gb0, sem0).wait()
    pltpu.sync_copy(gb0, agg_s.at[dst_v.at[NBLK - 2]], add=True)
    pltpu.make_async_copy(y_hbm.at[src_v.at[NBLK - 1]], gb1, sem1).wait()
    pltpu.sync_copy(gb1, agg_s.at[dst_v.at[NBLK - 1]], add=True)
    plsc.subcore_barrier()

    wbase = s * ROWS_PER_TILE
    pltpu.sync_copy(agg_s.at[pl.ds(wbase, ROWS_PER_TILE)],
                    out_hbm.at[c, pl.ds(wbase, ROWS_PER_TILE)])


R = 1000


def _dinv_from(deg_blk):
    deg = deg_blk[0, :, 0:1] + deg_blk[1, :, 0:1] + 1.0
    return lax.rsqrt(deg)


def _tc1_body(x_ref, deg_ref, pw_ref, pb_ref, c1w_ref,
              g_ref, b_ref, m_ref, v_ref, y1_ref):
    xb = x_ref[...]
    sc = g_ref[...] * lax.rsqrt(v_ref[...] + EPS)
    h0 = (xb - m_ref[...]) * sc + b_ref[...]
    h1 = jnp.maximum(
        jnp.dot(h0, pw_ref[...], preferred_element_type=jnp.float32)
        + pb_ref[...], 0.0)
    xw = jnp.dot(h1, c1w_ref[...], preferred_element_type=jnp.float32)
    y1_ref[...] = xw * _dinv_from(deg_ref[...])


def _tc2_body(parts_ref, y1_ref, deg_ref, c1b_ref,
              g_ref, b_ref, m_ref, v_ref, c2w_ref, y2_ref):
    p = parts_ref[...]
    dinv = _dinv_from(deg_ref[...])
    agg = (p[0] + p[1] + y1_ref[...]) * dinv + c1b_ref[...]
    sc = g_ref[...] * lax.rsqrt(v_ref[...] + EPS)
    u = jnp.maximum((agg - m_ref[...]) * sc + b_ref[...], 0.0)
    y2_ref[...] = jnp.dot(
        u, c2w_ref[...], preferred_element_type=jnp.float32) * dinv


def _tc3_body(parts_ref, y2_ref, deg_ref, c2b_ref,
              g_ref, b_ref, m_ref, v_ref, out_ref):
    p = parts_ref[...]
    dinv = _dinv_from(deg_ref[...])
    agg = (p[0] + p[1] + y2_ref[...]) * dinv + c2b_ref[...]
    sc = g_ref[...] * lax.rsqrt(v_ref[...] + EPS)
    out_ref[...] = (agg - m_ref[...]) * sc + b_ref[...]


def _row_spec():
    return pl.BlockSpec((R, DD), lambda i: (i, 0))


def _parts_spec():
    return pl.BlockSpec((NC, R, DD), lambda i: (0, i, 0))


def _deg_spec():
    return pl.BlockSpec((NC, R, DEGW), lambda i: (0, i, 0))


def _mat_spec():
    return pl.BlockSpec((DD, DD), lambda i: (0, 0))


def _vec_spec():
    return pl.BlockSpec((1, DD), lambda i: (0, 0))


def _f32_out():
    return jax.ShapeDtypeStruct((NN, DD), jnp.float32)


def kernel(x, edge_index, proj_W, proj_b, conv1_W, conv1_b, conv2_W, conv2_b,
           bn_in_g, bn_in_b, bn_in_m, bn_in_v,
           bn1_g, bn1_b, bn1_m, bn1_v,
           bn2_g, bn2_b, bn2_m, bn2_v):
    e = edge_index.shape[1]
    pad = CAP - e
    srcp = jnp.concatenate(
        [edge_index[0], jnp.zeros((pad,), jnp.int32)]).reshape(NC, NS, NBLK, BLK)
    dstp = jnp.concatenate(
        [edge_index[1], jnp.full((pad,), NN, jnp.int32)]).reshape(NC, NS, NBLK, BLK)

    v2 = lambda a: a.reshape(1, DD)

    deg_parts = _deg_kernel(dstp)

    y1 = pl.pallas_call(
        _tc1_body,
        grid=(NN // R,),
        in_specs=[_row_spec(), _deg_spec(), _mat_spec(), _vec_spec(),
                  _mat_spec(), _vec_spec(), _vec_spec(), _vec_spec(),
                  _vec_spec()],
        out_specs=_row_spec(),
        out_shape=_f32_out(),
    )(x, deg_parts, proj_W, v2(proj_b), conv1_W,
      v2(bn_in_g), v2(bn_in_b), v2(bn_in_m), v2(bn_in_v))

    parts1 = _agg_kernel(y1, srcp, dstp)

    y2 = pl.pallas_call(
        _tc2_body,
        grid=(NN // R,),
        in_specs=[_parts_spec(), _row_spec(), _deg_spec(), _vec_spec(),
                  _vec_spec(), _vec_spec(), _vec_spec(), _vec_spec(),
                  _mat_spec()],
        out_specs=_row_spec(),
        out_shape=_f32_out(),
    )(parts1, y1, deg_parts, v2(conv1_b),
      v2(bn1_g), v2(bn1_b), v2(bn1_m), v2(bn1_v), conv2_W)

    parts2 = _agg_kernel(y2, srcp, dstp)

    out = pl.pallas_call(
        _tc3_body,
        grid=(NN // R,),
        in_specs=[_parts_spec(), _row_spec(), _deg_spec(), _vec_spec(),
                  _vec_spec(), _vec_spec(), _vec_spec(), _vec_spec()],
        out_specs=_row_spec(),
        out_shape=_f32_out(),
    )(parts2, y2, deg_parts, v2(conv2_b),
      v2(bn2_g), v2(bn2_b), v2(bn2_m), v2(bn2_v))

    return out


# speedup vs baseline: 7.0984x; 7.0984x over previous
import functools

import jax
import jax.numpy as jnp
from jax import lax
from jax.experimental import pallas as pl
from jax.experimental.pallas import tpu as pltpu
from jax.experimental.pallas import tpu_sc as plsc

NN = 10000
DD = 128
EPS = 1e-5

NC = 2
NS = 16
BLK = 64
CB = 16
NCH = 10
EPT = NCH * CB * BLK
CAP = NC * NS * EPT
NPAD = 10240
WB = 624
ZROWS = 16
DEGW = 16

_mesh = plsc.VectorSubcoreMesh(
    core_axis_name="c", subcore_axis_name="s", num_cores=NC, num_subcores=NS)


@functools.partial(
    pl.kernel,
    out_type=jax.ShapeDtypeStruct((NC, NN, DEGW), jnp.float32),
    mesh=_mesh,
    scratch_types=[
        pltpu.VMEM_SHARED((NPAD, DEGW), jnp.float32),
        pltpu.VMEM((CB, BLK), jnp.int32),
        pltpu.VMEM((BLK, DEGW), jnp.float32),
        pltpu.VMEM((BLK, DEGW), jnp.float32),
    ],
)
def _deg_kernel(slab_hbm, out_hbm, deg_s, dst_v, obuf, zbuf):
    c = lax.axis_index("c")
    s = lax.axis_index("s")
    ones16 = jnp.full((16,), 1.0, jnp.float32)
    zeros16 = jnp.zeros((16,), jnp.float32)

    def fill(i, _):
        obuf[i, :] = ones16
        zbuf[i, :] = zeros16
        return 0
    lax.fori_loop(0, BLK, fill, 0)

    rows_pt = NPAD // NS
    zbase = s * rows_pt

    def zloop(i, _):
        pltpu.sync_copy(zbuf, deg_s.at[pl.ds(zbase + i * BLK, BLK)])
        return 0
    lax.fori_loop(0, rows_pt // BLK, zloop, 0)
    plsc.subcore_barrier()

    def chloop(ch, _):
        pltpu.sync_copy(slab_hbm.at[c, s, ch, 1], dst_v)

        def sloop(j, _):
            pltpu.sync_copy(obuf, deg_s.at[dst_v.at[j]], add=True)
            return 0
        lax.fori_loop(0, CB, sloop, 0)
        return 0
    lax.fori_loop(0, NCH, chloop, 0)
    plsc.subcore_barrier()

    wbase = s * WB

    def wloop(k, _):
        b = wbase + k * BLK
        pltpu.sync_copy(deg_s.at[pl.ds(b, BLK)], zbuf)
        pltpu.sync_copy(zbuf, out_hbm.at[c, pl.ds(b, BLK)])
        return 0
    lax.fori_loop(0, WB // BLK, wloop, 0)
    pltpu.sync_copy(deg_s.at[pl.ds(wbase + 576, 48)], zbuf.at[pl.ds(0, 48)])
    pltpu.sync_copy(zbuf.at[pl.ds(0, 48)], out_hbm.at[c, pl.ds(wbase + 576, 48)])

    @pl.when(s == NS - 1)
    def _():
        pltpu.sync_copy(deg_s.at[pl.ds(NS * WB, NN - NS * WB)],
                        zbuf.at[pl.ds(0, NN - NS * WB)])
        pltpu.sync_copy(zbuf.at[pl.ds(0, NN - NS * WB)],
                        out_hbm.at[c, pl.ds(NS * WB, NN - NS * WB)])


@functools.partial(
    pl.kernel,
    out_type=jax.ShapeDtypeStruct((NC, NN, DD), jnp.float32),
    mesh=_mesh,
    scratch_types=[
        pltpu.VMEM_SHARED((NPAD, DD), jnp.float32),
        pltpu.VMEM((2, CB, BLK), jnp.int32),
        pltpu.VMEM((2, CB, BLK), jnp.int32),
        pltpu.VMEM((BLK, DD), jnp.float32),
        pltpu.VMEM((BLK, DD), jnp.float32),
        pltpu.VMEM((ZROWS, DD), jnp.float32),
        pltpu.SemaphoreType.DMA,
        pltpu.SemaphoreType.DMA,
        pltpu.SemaphoreType.DMA,
        pltpu.SemaphoreType.DMA,
    ],
)
def _agg_kernel(y_hbm, slab_hbm, out_hbm,
                agg_s, w0, w1, gb0, gb1, zbuf, sem0, sem1, isem0, isem1):
    c = lax.axis_index("c")
    s = lax.axis_index("s")
    zeros16 = jnp.zeros((16,), jnp.float32)
    lanes_per_row = DD // 16

    def zfill(i, _):
        r = i // lanes_per_row
        k = (i % lanes_per_row) * 16
        zbuf[r, pl.ds(k, 16)] = zeros16
        return 0
    lax.fori_loop(0, ZROWS * lanes_per_row, zfill, 0)

    rows_pt = NPAD // NS
    zbase = s * rows_pt

    def zloop(i, _):
        pltpu.sync_copy(zbuf, agg_s.at[pl.ds(zbase + i * ZROWS, ZROWS)])
        return 0
    lax.fori_loop(0, rows_pt // ZROWS, zloop, 0)
    plsc.subcore_barrier()

    def chloop(ch, _):
        pltpu.sync_copy(slab_hbm.at[c, s, ch], w0)

        def bloop(j, _):
            pltpu.async_copy(y_hbm.at[w0.at[0, j]], gb0, sem0).wait()
            pltpu.sync_copy(gb0, agg_s.at[w0.at[1, j]], add=True)
            return 0
        lax.fori_loop(0, CB, bloop, 0)
        return 0
    lax.fori_loop(0, NCH, chloop, 0)
    plsc.subcore_barrier()

    wbase = s * WB

    def wloop(k, _):
        b = wbase + k * BLK
        pltpu.sync_copy(agg_s.at[pl.ds(b, BLK)], gb0)
        pltpu.sync_copy(gb0, out_hbm.at[c, pl.ds(b, BLK)])
        return 0
    lax.fori_loop(0, WB // BLK, wloop, 0)
    pltpu.sync_copy(agg_s.at[pl.ds(wbase + 576, 48)], gb0.at[pl.ds(0, 48)])
    pltpu.sync_copy(gb0.at[pl.ds(0, 48)], out_hbm.at[c, pl.ds(wbase + 576, 48)])

    @pl.when(s == NS - 1)
    def _():
        pltpu.sync_copy(agg_s.at[pl.ds(NS * WB, NN - NS * WB)],
                        gb0.at[pl.ds(0, NN - NS * WB)])
        pltpu.sync_copy(gb0.at[pl.ds(0, NN - NS * WB)],
                        out_hbm.at[c, pl.ds(NS * WB, NN - NS * WB)])


R = 1000


def _dinv_from(deg_blk):
    deg = deg_blk[0, :, 0:1] + deg_blk[1, :, 0:1] + 1.0
    return lax.rsqrt(deg)


def _tc1_body(x_ref, deg_ref, pw_ref, pb_ref, c1w_ref,
              g_ref, b_ref, m_ref, v_ref, y1_ref):
    xb = x_ref[...]
    sc = g_ref[...] * lax.rsqrt(v_ref[...] + EPS)
    h0 = (xb - m_ref[...]) * sc + b_ref[...]
    h1 = jnp.maximum(
        jnp.dot(h0, pw_ref[...], preferred_element_type=jnp.float32)
        + pb_ref[...], 0.0)
    xw = jnp.dot(h1, c1w_ref[...], preferred_element_type=jnp.float32)
    y1_ref[...] = xw * _dinv_from(deg_ref[...])


def _tc2_body(parts_ref, y1_ref, deg_ref, c1b_ref,
              g_ref, b_ref, m_ref, v_ref, c2w_ref, y2_ref):
    p = parts_ref[...]
    dinv = _dinv_from(deg_ref[...])
    agg = (p[0] + p[1] + y1_ref[...]) * dinv + c1b_ref[...]
    sc = g_ref[...] * lax.rsqrt(v_ref[...] + EPS)
    u = jnp.maximum((agg - m_ref[...]) * sc + b_ref[...], 0.0)
    y2_ref[...] = jnp.dot(
        u, c2w_ref[...], preferred_element_type=jnp.float32) * dinv


def _tc3_body(parts_ref, y2_ref, deg_ref, c2b_ref,
              g_ref, b_ref, m_ref, v_ref, out_ref):
    p = parts_ref[...]
    dinv = _dinv_from(deg_ref[...])
    agg = (p[0] + p[1] + y2_ref[...]) * dinv + c2b_ref[...]
    sc = g_ref[...] * lax.rsqrt(v_ref[...] + EPS)
    out_ref[...] = (agg - m_ref[...]) * sc + b_ref[...]


def _row_spec():
    return pl.BlockSpec((R, DD), lambda i: (i, 0))


def _parts_spec():
    return pl.BlockSpec((NC, R, DD), lambda i: (0, i, 0))


def _deg_spec():
    return pl.BlockSpec((NC, R, DEGW), lambda i: (0, i, 0))


def _mat_spec():
    return pl.BlockSpec((DD, DD), lambda i: (0, 0))


def _vec_spec():
    return pl.BlockSpec((1, DD), lambda i: (0, 0))


def _f32_out():
    return jax.ShapeDtypeStruct((NN, DD), jnp.float32)


def kernel(x, edge_index, proj_W, proj_b, conv1_W, conv1_b, conv2_W, conv2_b,
           bn_in_g, bn_in_b, bn_in_m, bn_in_v,
           bn1_g, bn1_b, bn1_m, bn1_v,
           bn2_g, bn2_b, bn2_m, bn2_v):
    e = edge_index.shape[1]
    pad = CAP - e
    srcp = jnp.concatenate(
        [edge_index[0], jnp.zeros((pad,), jnp.int32)]
    ).reshape(NC, NS, NCH, CB, BLK)
    dstp = jnp.concatenate(
        [edge_index[1], jnp.full((pad,), NN, jnp.int32)]
    ).reshape(NC, NS, NCH, CB, BLK)
    slab = jnp.stack([srcp, dstp], axis=3)

    v2 = lambda a: a.reshape(1, DD)

    deg_parts = _deg_kernel(slab)

    y1 = pl.pallas_call(
        _tc1_body,
        grid=(NN // R,),
        in_specs=[_row_spec(), _deg_spec(), _mat_spec(), _vec_spec(),
                  _mat_spec(), _vec_spec(), _vec_spec(), _vec_spec(),
                  _vec_spec()],
        out_specs=_row_spec(),
        out_shape=_f32_out(),
    )(x, deg_parts, proj_W, v2(proj_b), conv1_W,
      v2(bn_in_g), v2(bn_in_b), v2(bn_in_m), v2(bn_in_v))

    parts1 = _agg_kernel(y1, slab)

    y2 = pl.pallas_call(
        _tc2_body,
        grid=(NN // R,),
        in_specs=[_parts_spec(), _row_spec(), _deg_spec(), _vec_spec(),
                  _vec_spec(), _vec_spec(), _vec_spec(), _vec_spec(),
                  _mat_spec()],
        out_specs=_row_spec(),
        out_shape=_f32_out(),
    )(parts1, y1, deg_parts, v2(conv1_b),
      v2(bn1_g), v2(bn1_b), v2(bn1_m), v2(bn1_v), conv2_W)

    parts2 = _agg_kernel(y2, slab)

    out = pl.pallas_call(
        _tc3_body,
        grid=(NN // R,),
        in_specs=[_parts_spec(), _row_spec(), _deg_spec(), _vec_spec(),
                  _vec_spec(), _vec_spec(), _vec_spec(), _vec_spec()],
        out_specs=_row_spec(),
        out_shape=_f32_out(),
    )(parts2, y2, deg_parts, v2(conv2_b),
      v2(bn2_g), v2(bn2_b), v2(bn2_m), v2(bn2_v))

    return out


# speedup vs baseline: 8.3870x; 1.1815x over previous
import functools

import jax
import jax.numpy as jnp
from jax import lax
from jax.experimental import pallas as pl
from jax.experimental.pallas import tpu as pltpu
from jax.experimental.pallas import tpu_sc as plsc

NN = 10000
DD = 128
EPS = 1e-5

NC = 2
NS = 16
BLK = 64
CB = 16
NCH = 10
EPT = NCH * CB * BLK
CAP = NC * NS * EPT
NPAD = 10240
WB = 624
ZROWS = 16
DEGW = 16

_mesh = plsc.VectorSubcoreMesh(
    core_axis_name="c", subcore_axis_name="s", num_cores=NC, num_subcores=NS)


@functools.partial(
    pl.kernel,
    out_type=jax.ShapeDtypeStruct((NC, NN, DEGW), jnp.float32),
    mesh=_mesh,
    scratch_types=[
        pltpu.VMEM_SHARED((NPAD, DEGW), jnp.float32),
        pltpu.VMEM((CB, BLK), jnp.int32),
        pltpu.VMEM((BLK, DEGW), jnp.float32),
        pltpu.VMEM((BLK, DEGW), jnp.float32),
    ],
)
def _deg_kernel(slab_hbm, out_hbm, deg_s, dst_v, obuf, zbuf):
    c = lax.axis_index("c")
    s = lax.axis_index("s")
    ones16 = jnp.full((16,), 1.0, jnp.float32)
    zeros16 = jnp.zeros((16,), jnp.float32)

    def fill(i, _):
        obuf[i, :] = ones16
        zbuf[i, :] = zeros16
        return 0
    lax.fori_loop(0, BLK, fill, 0)

    rows_pt = NPAD // NS
    zbase = s * rows_pt

    def zloop(i, _):
        pltpu.sync_copy(zbuf, deg_s.at[pl.ds(zbase + i * BLK, BLK)])
        return 0
    lax.fori_loop(0, rows_pt // BLK, zloop, 0)
    plsc.subcore_barrier()

    def chloop(ch, _):
        pltpu.sync_copy(slab_hbm.at[c, s, ch, 1], dst_v)

        def sloop(j, _):
            pltpu.sync_copy(obuf, deg_s.at[dst_v.at[j]], add=True)
            return 0
        lax.fori_loop(0, CB, sloop, 0)
        return 0
    lax.fori_loop(0, NCH, chloop, 0)
    plsc.subcore_barrier()

    wbase = s * WB

    def wloop(k, _):
        b = wbase + k * BLK
        pltpu.sync_copy(deg_s.at[pl.ds(b, BLK)], zbuf)
        pltpu.sync_copy(zbuf, out_hbm.at[c, pl.ds(b, BLK)])
        return 0
    lax.fori_loop(0, WB // BLK, wloop, 0)
    pltpu.sync_copy(deg_s.at[pl.ds(wbase + 576, 48)], zbuf.at[pl.ds(0, 48)])
    pltpu.sync_copy(zbuf.at[pl.ds(0, 48)], out_hbm.at[c, pl.ds(wbase + 576, 48)])

    @pl.when(s == NS - 1)
    def _():
        pltpu.sync_copy(deg_s.at[pl.ds(NS * WB, NN - NS * WB)],
                        zbuf.at[pl.ds(0, NN - NS * WB)])
        pltpu.sync_copy(zbuf.at[pl.ds(0, NN - NS * WB)],
                        out_hbm.at[c, pl.ds(NS * WB, NN - NS * WB)])


@functools.partial(
    pl.kernel,
    out_type=jax.ShapeDtypeStruct((NC, NN, DD), jnp.float32),
    mesh=_mesh,
    scratch_types=[
        pltpu.VMEM_SHARED((NPAD, DD), jnp.float32),
        pltpu.VMEM((2, CB, BLK), jnp.int32),
        pltpu.VMEM((2, CB, BLK), jnp.int32),
        pltpu.VMEM((BLK, DD), jnp.float32),
        pltpu.VMEM((BLK, DD), jnp.float32),
        pltpu.VMEM((BLK, DD), jnp.float32),
        pltpu.VMEM((ZROWS, DD), jnp.float32),
        pltpu.SemaphoreType.DMA,
        pltpu.SemaphoreType.DMA,
        pltpu.SemaphoreType.DMA,
        pltpu.SemaphoreType.DMA,
        pltpu.SemaphoreType.DMA,
        pltpu.SemaphoreType.DMA,
        pltpu.SemaphoreType.DMA,
        pltpu.SemaphoreType.DMA,
    ],
)
def _agg_kernel(y_hbm, slab_hbm, out_hbm,
                agg_s, w0, w1, gb0, gb1, gb2, zbuf,
                gsem0, gsem1, gsem2, ssem0, ssem1, ssem2, isem0, isem1):
    c = lax.axis_index("c")
    s = lax.axis_index("s")
    zeros16 = jnp.zeros((16,), jnp.float32)
    lanes_per_row = DD // 16

    def zfill(i, _):
        r = i // lanes_per_row
        k = (i % lanes_per_row) * 16
        zbuf[r, pl.ds(k, 16)] = zeros16
        return 0
    lax.fori_loop(0, ZROWS * lanes_per_row, zfill, 0)

    rows_pt = NPAD // NS
    zbase = s * rows_pt

    def zloop(i, _):
        pltpu.sync_copy(zbuf, agg_s.at[pl.ds(zbase + i * ZROWS, ZROWS)])
        return 0
    lax.fori_loop(0, rows_pt // ZROWS, zloop, 0)
    plsc.subcore_barrier()

    wbufs = (w0, w1)
    isems = (isem0, isem1)
    gbufs = (gb0, gb1, gb2)
    gsems = (gsem0, gsem1, gsem2)
    ssems = (ssem0, ssem1, ssem2)
    NBUF = 3
    G = 2
    TOT = NCH * CB
    idx_d = [None] * NCH
    gd = [None] * TOT
    sd = [None] * TOT
    idx_d[0] = pltpu.async_copy(slab_hbm.at[c, s, 0], w0, isem0)
    for t in range(TOT + G):
        if t < TOT:
            ch, j = divmod(t, CB)
            if j == 0:
                idx_d[ch].wait()
            if j == NBUF and ch + 1 < NCH:
                idx_d[ch + 1] = pltpu.async_copy(
                    slab_hbm.at[c, s, ch + 1], wbufs[(ch + 1) % 2],
                    isems[(ch + 1) % 2])
            b = t % NBUF
            if t >= NBUF:
                sd[t - NBUF].wait()
            gd[t] = pltpu.async_copy(
                y_hbm.at[wbufs[ch % 2].at[0, j]], gbufs[b], gsems[b])
        u = t - G
        if 0 <= u < TOT:
            cu, ju = divmod(u, CB)
            gd[u].wait()
            sd[u] = pltpu.async_copy(
                gbufs[u % NBUF], agg_s.at[wbufs[cu % 2].at[1, ju]],
                ssems[u % NBUF], add=True)
    for u in range(TOT - NBUF, TOT):
        sd[u].wait()
    plsc.subcore_barrier()

    wbase = s * WB

    def wloop(k, _):
        b = wbase + k * BLK
        pltpu.sync_copy(agg_s.at[pl.ds(b, BLK)], gb0)
        pltpu.sync_copy(gb0, out_hbm.at[c, pl.ds(b, BLK)])
        return 0
    lax.fori_loop(0, WB // BLK, wloop, 0)
    pltpu.sync_copy(agg_s.at[pl.ds(wbase + 576, 48)], gb0.at[pl.ds(0, 48)])
    pltpu.sync_copy(gb0.at[pl.ds(0, 48)], out_hbm.at[c, pl.ds(wbase + 576, 48)])

    @pl.when(s == NS - 1)
    def _():
        pltpu.sync_copy(agg_s.at[pl.ds(NS * WB, NN - NS * WB)],
                        gb0.at[pl.ds(0, NN - NS * WB)])
        pltpu.sync_copy(gb0.at[pl.ds(0, NN - NS * WB)],
                        out_hbm.at[c, pl.ds(NS * WB, NN - NS * WB)])


R = 1000


def _dinv_from(deg_blk):
    deg = deg_blk[0, :, 0:1] + deg_blk[1, :, 0:1] + 1.0
    return lax.rsqrt(deg)


def _tc1_body(x_ref, deg_ref, pw_ref, pb_ref, c1w_ref,
              g_ref, b_ref, m_ref, v_ref, y1_ref):
    xb = x_ref[...]
    sc = g_ref[...] * lax.rsqrt(v_ref[...] + EPS)
    h0 = (xb - m_ref[...]) * sc + b_ref[...]
    h1 = jnp.maximum(
        jnp.dot(h0, pw_ref[...], preferred_element_type=jnp.float32)
        + pb_ref[...], 0.0)
    xw = jnp.dot(h1, c1w_ref[...], preferred_element_type=jnp.float32)
    y1_ref[...] = xw * _dinv_from(deg_ref[...])


def _tc2_body(parts_ref, y1_ref, deg_ref, c1b_ref,
              g_ref, b_ref, m_ref, v_ref, c2w_ref, y2_ref):
    p = parts_ref[...]
    dinv = _dinv_from(deg_ref[...])
    agg = (p[0] + p[1] + y1_ref[...]) * dinv + c1b_ref[...]
    sc = g_ref[...] * lax.rsqrt(v_ref[...] + EPS)
    u = jnp.maximum((agg - m_ref[...]) * sc + b_ref[...], 0.0)
    y2_ref[...] = jnp.dot(
        u, c2w_ref[...], preferred_element_type=jnp.float32) * dinv


def _tc3_body(parts_ref, y2_ref, deg_ref, c2b_ref,
              g_ref, b_ref, m_ref, v_ref, out_ref):
    p = parts_ref[...]
    dinv = _dinv_from(deg_ref[...])
    agg = (p[0] + p[1] + y2_ref[...]) * dinv + c2b_ref[...]
    sc = g_ref[...] * lax.rsqrt(v_ref[...] + EPS)
    out_ref[...] = (agg - m_ref[...]) * sc + b_ref[...]


def _row_spec():
    return pl.BlockSpec((R, DD), lambda i: (i, 0))


def _parts_spec():
    return pl.BlockSpec((NC, R, DD), lambda i: (0, i, 0))


def _deg_spec():
    return pl.BlockSpec((NC, R, DEGW), lambda i: (0, i, 0))


def _mat_spec():
    return pl.BlockSpec((DD, DD), lambda i: (0, 0))


def _vec_spec():
    return pl.BlockSpec((1, DD), lambda i: (0, 0))


def _f32_out():
    return jax.ShapeDtypeStruct((NN, DD), jnp.float32)


def kernel(x, edge_index, proj_W, proj_b, conv1_W, conv1_b, conv2_W, conv2_b,
           bn_in_g, bn_in_b, bn_in_m, bn_in_v,
           bn1_g, bn1_b, bn1_m, bn1_v,
           bn2_g, bn2_b, bn2_m, bn2_v):
    e = edge_index.shape[1]
    pad = CAP - e
    srcp = jnp.concatenate(
        [edge_index[0], jnp.zeros((pad,), jnp.int32)]
    ).reshape(NC, NS, NCH, CB, BLK)
    dstp = jnp.concatenate(
        [edge_index[1], jnp.full((pad,), NN, jnp.int32)]
    ).reshape(NC, NS, NCH, CB, BLK)
    slab = jnp.stack([srcp, dstp], axis=3)

    v2 = lambda a: a.reshape(1, DD)

    deg_parts = _deg_kernel(slab)

    y1 = pl.pallas_call(
        _tc1_body,
        grid=(NN // R,),
        in_specs=[_row_spec(), _deg_spec(), _mat_spec(), _vec_spec(),
                  _mat_spec(), _vec_spec(), _vec_spec(), _vec_spec(),
                  _vec_spec()],
        out_specs=_row_spec(),
        out_shape=_f32_out(),
    )(x, deg_parts, proj_W, v2(proj_b), conv1_W,
      v2(bn_in_g), v2(bn_in_b), v2(bn_in_m), v2(bn_in_v))

    parts1 = _agg_kernel(y1, slab)

    y2 = pl.pallas_call(
        _tc2_body,
        grid=(NN // R,),
        in_specs=[_parts_spec(), _row_spec(), _deg_spec(), _vec_spec(),
                  _vec_spec(), _vec_spec(), _vec_spec(), _vec_spec(),
                  _mat_spec()],
        out_specs=_row_spec(),
        out_shape=_f32_out(),
    )(parts1, y1, deg_parts, v2(conv1_b),
      v2(bn1_g), v2(bn1_b), v2(bn1_m), v2(bn1_v), conv2_W)

    parts2 = _agg_kernel(y2, slab)

    out = pl.pallas_call(
        _tc3_body,
        grid=(NN // R,),
        in_specs=[_parts_spec(), _row_spec(), _deg_spec(), _vec_spec(),
                  _vec_spec(), _vec_spec(), _vec_spec(), _vec_spec()],
        out_specs=_row_spec(),
        out_shape=_f32_out(),
    )(parts2, y2, deg_parts, v2(conv2_b),
      v2(bn2_g), v2(bn2_b), v2(bn2_m), v2(bn2_v))

    return out


# speedup vs baseline: 8.7495x; 1.0432x over previous
import functools

import jax
import jax.numpy as jnp
from jax import lax
from jax.experimental import pallas as pl
from jax.experimental.pallas import tpu as pltpu
from jax.experimental.pallas import tpu_sc as plsc

NN = 10000
DD = 128
EPS = 1e-5

NC = 2
NS = 16
BLK = 64
CB = 16
K0 = 16
K1 = 4
NCHT = K0 + K1
ECH = CB * BLK
CAP = NS * NCHT * ECH
NPAD = 10112
WB = 624
ZROWS = 8
DEGW = 16

_mesh = plsc.VectorSubcoreMesh(
    core_axis_name="c", subcore_axis_name="s", num_cores=NC, num_subcores=NS)


def _chunks(total, step):
    out, b = [], 0
    while b < total:
        n = min(step, total - b)
        out.append((b, n))
        b += n
    return out


@functools.partial(
    pl.kernel,
    out_type=jax.ShapeDtypeStruct((NC, NN, DEGW), jnp.float32),
    mesh=_mesh,
    scratch_types=[
        pltpu.VMEM_SHARED((NPAD, DEGW), jnp.float32),
        pltpu.VMEM((2, CB, BLK), jnp.int32),
        pltpu.VMEM((BLK, DEGW), jnp.float32),
        pltpu.VMEM((BLK, DEGW), jnp.float32),
    ],
)
def _deg_kernel(slab_hbm, out_hbm, deg_s, w0, obuf, zbuf):
    c = lax.axis_index("c")
    s = lax.axis_index("s")
    ones16 = jnp.full((16,), 1.0, jnp.float32)
    zeros16 = jnp.zeros((16,), jnp.float32)

    def fill(i, _):
        obuf[i, :] = ones16
        zbuf[i, :] = zeros16
        return 0
    lax.fori_loop(0, BLK, fill, 0)

    rows_pt = NPAD // NS
    zbase = s * rows_pt
    for (b, n) in _chunks(rows_pt, BLK):
        pltpu.sync_copy(zbuf.at[pl.ds(0, n)],
                        deg_s.at[pl.ds(zbase + b, n)])
    plsc.subcore_barrier()

    def deg_chunks(count, start):
        def chloop(ch, _):
            pltpu.sync_copy(slab_hbm.at[start + ch], w0)

            def sloop(j, _):
                pltpu.sync_copy(obuf, deg_s.at[w0.at[1, j]], add=True)
                return 0
            lax.fori_loop(0, CB, sloop, 0)
            return 0
        lax.fori_loop(0, count, chloop, 0)

    @pl.when(c == 0)
    def _():
        deg_chunks(K0, s * K0)

    @pl.when(c == 1)
    def _():
        deg_chunks(K1, NS * K0 + s * K1)
    plsc.subcore_barrier()

    wbase = s * WB
    for (b, n) in _chunks(WB, BLK):
        pltpu.sync_copy(deg_s.at[pl.ds(wbase + b, n)], zbuf.at[pl.ds(0, n)])
        pltpu.sync_copy(zbuf.at[pl.ds(0, n)],
                        out_hbm.at[c, pl.ds(wbase + b, n)])

    @pl.when(s == NS - 1)
    def _():
        pltpu.sync_copy(deg_s.at[pl.ds(NS * WB, NN - NS * WB)],
                        zbuf.at[pl.ds(0, NN - NS * WB)])
        pltpu.sync_copy(zbuf.at[pl.ds(0, NN - NS * WB)],
                        out_hbm.at[c, pl.ds(NS * WB, NN - NS * WB)])


@functools.partial(
    pl.kernel,
    out_type=jax.ShapeDtypeStruct((NC, NN, DD), jnp.float32),
    mesh=_mesh,
    scratch_types=[
        pltpu.VMEM_SHARED((NPAD, DD), jnp.float32),
        pltpu.VMEM((2, CB, BLK), jnp.int32),
        pltpu.VMEM((2, CB, BLK), jnp.int32),
        pltpu.VMEM((BLK, DD), jnp.float32),
        pltpu.VMEM((BLK, DD), jnp.float32),
        pltpu.VMEM((ZROWS, DD), jnp.float32),
        pltpu.SemaphoreType.DMA,
        pltpu.SemaphoreType.DMA,
        pltpu.SemaphoreType.DMA,
        pltpu.SemaphoreType.DMA,
        pltpu.SemaphoreType.DMA,
        pltpu.SemaphoreType.DMA,
    ],
)
def _agg_kernel(y_hbm, slab_hbm, out_hbm,
                agg_s, w0, w1, gb0, gb1, zbuf,
                gsem0, gsem1, ssem0, ssem1, isem0, isem1):
    c = lax.axis_index("c")
    s = lax.axis_index("s")
    zeros16 = jnp.zeros((16,), jnp.float32)
    lanes_per_row = DD // 16

    def zfill(i, _):
        r = i // lanes_per_row
        k = (i % lanes_per_row) * 16
        zbuf[r, pl.ds(k, 16)] = zeros16
        return 0
    lax.fori_loop(0, ZROWS * lanes_per_row, zfill, 0)

    rows_pt = NPAD // NS
    zbase = s * rows_pt

    def zloop(i, _):
        pltpu.sync_copy(zbuf, agg_s.at[pl.ds(zbase + i * ZROWS, ZROWS)])
        return 0
    lax.fori_loop(0, rows_pt // ZROWS, zloop, 0)
    plsc.subcore_barrier()

    wbufs = (w0, w1)
    isems = (isem0, isem1)
    gbufs = (gb0, gb1)
    gsems = (gsem0, gsem1)
    ssems = (ssem0, ssem1)
    NBUF = 2
    G = 1

    def run_pipeline(nch, start_ch):
        tot = nch * CB
        idx_d = [None] * nch
        gd = [None] * tot
        sd = [None] * tot
        idx_d[0] = pltpu.async_copy(slab_hbm.at[start_ch], w0, isem0)
        for t in range(tot + G):
            if t < tot:
                ch, j = divmod(t, CB)
                if j == 0:
                    idx_d[ch].wait()
                if j == NBUF and ch + 1 < nch:
                    idx_d[ch + 1] = pltpu.async_copy(
                        slab_hbm.at[start_ch + ch + 1],
                        wbufs[(ch + 1) % 2], isems[(ch + 1) % 2])
                b = t % NBUF
                if t >= NBUF:
                    sd[t - NBUF].wait()
                gd[t] = pltpu.async_copy(
                    y_hbm.at[wbufs[ch % 2].at[0, j]], gbufs[b], gsems[b])
            u = t - G
            if 0 <= u < tot:
                cu, ju = divmod(u, CB)
                gd[u].wait()
                sd[u] = pltpu.async_copy(
                    gbufs[u % NBUF], agg_s.at[wbufs[cu % 2].at[1, ju]],
                    ssems[u % NBUF], add=True)
        for u in range(tot - NBUF, tot):
            sd[u].wait()

    @pl.when(c == 0)
    def _():
        run_pipeline(K0, s * K0)

    @pl.when(c == 1)
    def _():
        run_pipeline(K1, NS * K0 + s * K1)
    plsc.subcore_barrier()

    wbase = s * WB
    for (b, n) in _chunks(WB, BLK):
        pltpu.sync_copy(agg_s.at[pl.ds(wbase + b, n)], gb0.at[pl.ds(0, n)])
        pltpu.sync_copy(gb0.at[pl.ds(0, n)],
                        out_hbm.at[c, pl.ds(wbase + b, n)])

    @pl.when(s == NS - 1)
    def _():
        pltpu.sync_copy(agg_s.at[pl.ds(NS * WB, NN - NS * WB)],
                        gb0.at[pl.ds(0, NN - NS * WB)])
        pltpu.sync_copy(gb0.at[pl.ds(0, NN - NS * WB)],
                        out_hbm.at[c, pl.ds(NS * WB, NN - NS * WB)])


R = 1000


def _dinv_from(deg_blk):
    deg = deg_blk[0, :, 0:1] + deg_blk[1, :, 0:1] + 1.0
    return lax.rsqrt(deg)


def _tc1_body(x_ref, deg_ref, pw_ref, pb_ref, c1w_ref,
              g_ref, b_ref, m_ref, v_ref, y1_ref):
    xb = x_ref[...]
    sc = g_ref[...] * lax.rsqrt(v_ref[...] + EPS)
    h0 = (xb - m_ref[...]) * sc + b_ref[...]
    h1 = jnp.maximum(
        jnp.dot(h0, pw_ref[...], preferred_element_type=jnp.float32)
        + pb_ref[...], 0.0)
    xw = jnp.dot(h1, c1w_ref[...], preferred_element_type=jnp.float32)
    y1_ref[...] = xw * _dinv_from(deg_ref[...])


def _tc2_body(parts_ref, y1_ref, deg_ref, c1b_ref,
              g_ref, b_ref, m_ref, v_ref, c2w_ref, y2_ref):
    p = parts_ref[...]
    dinv = _dinv_from(deg_ref[...])
    agg = (p[0] + p[1] + y1_ref[...]) * dinv + c1b_ref[...]
    sc = g_ref[...] * lax.rsqrt(v_ref[...] + EPS)
    u = jnp.maximum((agg - m_ref[...]) * sc + b_ref[...], 0.0)
    y2_ref[...] = jnp.dot(
        u, c2w_ref[...], preferred_element_type=jnp.float32) * dinv


def _tc3_body(parts_ref, y2_ref, deg_ref, c2b_ref,
              g_ref, b_ref, m_ref, v_ref, out_ref):
    p = parts_ref[...]
    dinv = _dinv_from(deg_ref[...])
    agg = (p[0] + p[1] + y2_ref[...]) * dinv + c2b_ref[...]
    sc = g_ref[...] * lax.rsqrt(v_ref[...] + EPS)
    out_ref[...] = (agg - m_ref[...]) * sc + b_ref[...]


def _row_spec():
    return pl.BlockSpec((R, DD), lambda i: (i, 0))


def _parts_spec():
    return pl.BlockSpec((NC, R, DD), lambda i: (0, i, 0))


def _deg_spec():
    return pl.BlockSpec((NC, R, DEGW), lambda i: (0, i, 0))


def _mat_spec():
    return pl.BlockSpec((DD, DD), lambda i: (0, 0))


def _vec_spec():
    return pl.BlockSpec((1, DD), lambda i: (0, 0))


def _f32_out():
    return jax.ShapeDtypeStruct((NN, DD), jnp.float32)


def kernel(x, edge_index, proj_W, proj_b, conv1_W, conv1_b, conv2_W, conv2_b,
           bn_in_g, bn_in_b, bn_in_m, bn_in_v,
           bn1_g, bn1_b, bn1_m, bn1_v,
           bn2_g, bn2_b, bn2_m, bn2_v):
    e = edge_index.shape[1]
    pad = CAP - e
    srcp = jnp.concatenate(
        [edge_index[0], jnp.zeros((pad,), jnp.int32)]
    ).reshape(NS * NCHT, CB, BLK)
    dstp = jnp.concatenate(
        [edge_index[1], jnp.full((pad,), NN, jnp.int32)]
    ).reshape(NS * NCHT, CB, BLK)
    slab = jnp.stack([srcp, dstp], axis=1)

    v2 = lambda a: a.reshape(1, DD)

    deg_parts = _deg_kernel(slab)

    y1 = pl.pallas_call(
        _tc1_body,
        grid=(NN // R,),
        in_specs=[_row_spec(), _deg_spec(), _mat_spec(), _vec_spec(),
                  _mat_spec(), _vec_spec(), _vec_spec(), _vec_spec(),
                  _vec_spec()],
        out_specs=_row_spec(),
        out_shape=_f32_out(),
    )(x, deg_parts, proj_W, v2(proj_b), conv1_W,
      v2(bn_in_g), v2(bn_in_b), v2(bn_in_m), v2(bn_in_v))

    parts1 = _agg_kernel(y1, slab)

    y2 = pl.pallas_call(
        _tc2_body,
        grid=(NN // R,),
        in_specs=[_parts_spec(), _row_spec(), _deg_spec(), _vec_spec(),
                  _vec_spec(), _vec_spec(), _vec_spec(), _vec_spec(),
                  _mat_spec()],
        out_specs=_row_spec(),
        out_shape=_f32_out(),
    )(parts1, y1, deg_parts, v2(conv1_b),
      v2(bn1_g), v2(bn1_b), v2(bn1_m), v2(bn1_v), conv2_W)

    parts2 = _agg_kernel(y2, slab)

    out = pl.pallas_call(
        _tc3_body,
        grid=(NN // R,),
        in_specs=[_parts_spec(), _row_spec(), _deg_spec(), _vec_spec(),
                  _vec_spec(), _vec_spec(), _vec_spec(), _vec_spec()],
        out_specs=_row_spec(),
        out_shape=_f32_out(),
    )(parts2, y2, deg_parts, v2(conv2_b),
      v2(bn2_g), v2(bn2_b), v2(bn2_m), v2(bn2_v))

    return out
